# baseline (speedup 1.0000x reference)
import jax
import jax.numpy as jnp
from jax import lax
from jax.experimental import pallas as pl
from jax.experimental.pallas import tpu as pltpu
from jax.experimental.pallas import tpu_sc as plsc

VOCAB = 100000
EMBED_DIM = 64
CONTEXT = 200
HIDDEN = 128
VTILE = 25000
NT = VOCAB // VTILE


_NC = 2
_BPW = 8
_ACTIVE = CONTEXT // _BPW


def _gather_body(idx_hbm, table_hbm, b2_hbm, out_hbm, b2out_hbm,
                 idx_v, grp_v, cols_v, b2_v, sem):
    wid = lax.axis_index("s") * _NC + lax.axis_index("c")

    @pl.when(wid >= 28)
    def _():
        r = wid - 28
        pltpu.sync_copy(b2_hbm.at[pl.ds(r * VTILE, VTILE)], b2_v)
        pltpu.sync_copy(b2_v, b2out_hbm.at[r])

    @pl.when(wid < _ACTIVE)
    def _():
        base = wid * _BPW
        sw = pl.multiple_of(jnp.minimum(base, CONTEXT - 16), 8)
        off = base - sw
        pltpu.sync_copy(idx_hbm.at[pl.ds(sw, 16)], idx_v)
        iota16 = lax.iota(jnp.int32, 16)

        def _fire(j, _):
            vj = plsc.load_gather(idx_v, [iota16 * 0 + (off + j)])[0]
            start = pl.multiple_of((vj // 128) * 128, 128)
            for d in range(EMBED_DIM // 8):
                pltpu.async_copy(
                    table_hbm.at[pl.ds(d * 8, 8), pl.ds(start, 128)],
                    grp_v.at[j, pl.ds(d * 8, 8)], sem)
            return 0

        lax.fori_loop(0, _BPW, _fire, 0)

        def _drain(j, _):
            pltpu.make_async_copy(
                table_hbm.at[pl.ds(0, 64), pl.ds(0, 128)], grp_v.at[j],
                sem).wait()
            return 0

        lax.fori_loop(0, _BPW, _drain, 0)

        def _select(j, _):
            vj = plsc.load_gather(idx_v, [iota16 * 0 + (off + j)])[0]
            r = iota16 * 0 + (vj % 128)
            jv = iota16 * 0 + j
            for a in range(EMBED_DIM // 16):
                rows = iota16 + a * 16
                val = plsc.load_gather(grp_v, [jv, rows, r])
                cols_v[pl.ds(j * EMBED_DIM + a * 16, 16)] = val
            return 0

        lax.fori_loop(0, _BPW, _select, 0)
        pltpu.sync_copy(cols_v,
                        out_hbm.at[0, pl.ds(wid * _BPW * EMBED_DIM,
                                            _BPW * EMBED_DIM)])


def _sc_gather(idx, table_t, b2):
    mesh = plsc.VectorSubcoreMesh(core_axis_name="c", subcore_axis_name="s")
    k = pl.kernel(
        _gather_body,
        mesh=mesh,
        out_type=(
            jax.ShapeDtypeStruct((1, CONTEXT * EMBED_DIM), jnp.float32),
            jax.ShapeDtypeStruct((NT, VTILE), jnp.float32),
        ),
        scratch_types=[
            pltpu.VMEM((16,), jnp.int32),
            pltpu.VMEM((_BPW, EMBED_DIM, 128), jnp.float32),
            pltpu.VMEM((_BPW * EMBED_DIM,), jnp.float32),
            pltpu.VMEM((VTILE,), jnp.float32),
            pltpu.SemaphoreType.DMA,
        ],
        compiler_params=pltpu.CompilerParams(needs_layout_passes=False),
    )
    return k(idx, table_t, b2)


def _mlp_body(emb_ref, w1_ref, b1_ref, w2_ref, b2_ref, out_ref, h_ref):
    i = pl.program_id(0)

    @pl.when(i == 0)
    def _():
        h = lax.dot_general(
            emb_ref[...], w1_ref[...], (((1,), (1,)), ((), ())),
            preferred_element_type=jnp.float32)
        h_ref[...] = jnp.maximum(h + b1_ref[...], 0.0)

    @pl.when(i < NT)
    def _():
        t = lax.dot_general(
            h_ref[...], w2_ref[...], (((1,), (1,)), ((), ())),
            preferred_element_type=jnp.float32)
        out_ref[pl.ds(i, 1), :] = t

    @pl.when(i == NT)
    def _():
        a = out_ref[...] + b2_ref[...]
        m = jnp.max(jnp.max(a, axis=1, keepdims=True), axis=0, keepdims=True)
        e = jnp.exp(a - m)
        s = jnp.sum(jnp.sum(e, axis=1, keepdims=True), axis=0, keepdims=True)
        out_ref[...] = a - (m + jnp.log(s))


def _mlp(emb_flat, W1, b1_2d, W2, b2_2d):
    return pl.pallas_call(
        _mlp_body,
        grid=(NT + 1,),
        in_specs=[
            pl.BlockSpec((1, CONTEXT * EMBED_DIM), lambda i: (0, 0)),
            pl.BlockSpec((HIDDEN, CONTEXT * EMBED_DIM), lambda i: (0, 0)),
            pl.BlockSpec((1, HIDDEN), lambda i: (0, 0)),
            pl.BlockSpec((VTILE, HIDDEN), lambda i: (jnp.minimum(i, NT - 1), 0)),
            pl.BlockSpec((NT, VTILE), lambda i: (0, 0)),
        ],
        out_specs=pl.BlockSpec((NT, VTILE), lambda i: (0, 0)),
        out_shape=jax.ShapeDtypeStruct((NT, VTILE), jnp.float32),
        scratch_shapes=[
            pltpu.VMEM((1, HIDDEN), jnp.float32),
        ],
    )(emb_flat, W1, b1_2d, W2, b2_2d)


def kernel(inputs, emb, W1, b1, W2, b2):
    emb_flat, b2_2d = _sc_gather(inputs, emb.T, b2)
    out = _mlp(emb_flat, W1, b1.reshape(1, HIDDEN), W2, b2_2d)
    return out.reshape(1, VOCAB)


# speedup vs baseline: 1.0032x; 1.0032x over previous
import jax
import jax.numpy as jnp
from jax import lax
from jax.experimental import pallas as pl
from jax.experimental.pallas import tpu as pltpu
from jax.experimental.pallas import tpu_sc as plsc

VOCAB = 100000
EMBED_DIM = 64
CONTEXT = 200
HIDDEN = 128
VTILE = 25000
NT = VOCAB // VTILE


_NC = 2
_BPW = 8
_ACTIVE = CONTEXT // _BPW


def _gather_body(idx_hbm, table_hbm, b2_hbm, out_hbm, b2out_hbm,
                 idx_v, grp_v, cols_v, b2_v, sem):
    wid = lax.axis_index("s") * _NC + lax.axis_index("c")

    @pl.when(wid >= 28)
    def _():
        r = wid - 28
        pltpu.sync_copy(b2_hbm.at[pl.ds(r * VTILE, VTILE)], b2_v)
        pltpu.sync_copy(b2_v, b2out_hbm.at[r])

    @pl.when(wid < _ACTIVE)
    def _():
        base = wid * _BPW
        sw = pl.multiple_of(jnp.minimum(base, CONTEXT - 16), 8)
        off = base - sw
        pltpu.sync_copy(idx_hbm.at[pl.ds(sw, 16)], idx_v)
        iota16 = lax.iota(jnp.int32, 16)

        def _fire(j, _):
            vj = plsc.load_gather(idx_v, [iota16 * 0 + (off + j)])[0]
            start = pl.multiple_of((vj // 128) * 128, 128)
            for d in range(EMBED_DIM // 8):
                pltpu.async_copy(
                    table_hbm.at[pl.ds(d * 8, 8), pl.ds(start, 128)],
                    grp_v.at[j, pl.ds(d * 8, 8)], sem)
            return 0

        lax.fori_loop(0, _BPW, _fire, 0)

        def _drain(j, _):
            pltpu.make_async_copy(
                table_hbm.at[pl.ds(0, 64), pl.ds(0, 128)], grp_v.at[j],
                sem).wait()
            return 0

        lax.fori_loop(0, _BPW, _drain, 0)

        def _select(j, _):
            vj = plsc.load_gather(idx_v, [iota16 * 0 + (off + j)])[0]
            r = iota16 * 0 + (vj % 128)
            jv = iota16 * 0 + j
            for a in range(EMBED_DIM // 16):
                rows = iota16 + a * 16
                val = plsc.load_gather(grp_v, [jv, rows, r])
                cols_v[pl.ds(j * EMBED_DIM + a * 16, 16)] = val
            return 0

        lax.fori_loop(0, _BPW, _select, 0)
        pltpu.sync_copy(cols_v,
                        out_hbm.at[0, pl.ds(wid * _BPW * EMBED_DIM,
                                            _BPW * EMBED_DIM)])


def _sc_gather(idx, table_t, b2):
    mesh = plsc.VectorSubcoreMesh(core_axis_name="c", subcore_axis_name="s")
    k = pl.kernel(
        _gather_body,
        mesh=mesh,
        out_type=(
            jax.ShapeDtypeStruct((1, CONTEXT * EMBED_DIM), jnp.float32),
            jax.ShapeDtypeStruct((NT, VTILE), jnp.float32),
        ),
        scratch_types=[
            pltpu.VMEM((16,), jnp.int32),
            pltpu.VMEM((_BPW, EMBED_DIM, 128), jnp.float32),
            pltpu.VMEM((_BPW * EMBED_DIM,), jnp.float32),
            pltpu.VMEM((VTILE,), jnp.float32),
            pltpu.SemaphoreType.DMA,
        ],
        compiler_params=pltpu.CompilerParams(needs_layout_passes=False,
                                             skip_device_barrier=True),
    )
    return k(idx, table_t, b2)


def _mlp_body(emb_ref, w1_ref, b1_ref, w2_ref, b2_ref, out_ref, h_ref):
    i = pl.program_id(0)

    @pl.when(i == 0)
    def _():
        h = lax.dot_general(
            emb_ref[...], w1_ref[...], (((1,), (1,)), ((), ())),
            preferred_element_type=jnp.float32)
        h_ref[...] = jnp.maximum(h + b1_ref[...], 0.0)

    @pl.when(i < NT)
    def _():
        t = lax.dot_general(
            h_ref[...], w2_ref[...], (((1,), (1,)), ((), ())),
            preferred_element_type=jnp.float32)
        out_ref[pl.ds(i, 1), :] = t

    @pl.when(i == NT)
    def _():
        a = out_ref[...] + b2_ref[...]
        m = jnp.max(jnp.max(a, axis=1, keepdims=True), axis=0, keepdims=True)
        e = jnp.exp(a - m)
        s = jnp.sum(jnp.sum(e, axis=1, keepdims=True), axis=0, keepdims=True)
        out_ref[...] = a - (m + jnp.log(s))


def _mlp(emb_flat, W1, b1_2d, W2, b2_2d):
    return pl.pallas_call(
        _mlp_body,
        grid=(NT + 1,),
        in_specs=[
            pl.BlockSpec((1, CONTEXT * EMBED_DIM), lambda i: (0, 0)),
            pl.BlockSpec((HIDDEN, CONTEXT * EMBED_DIM), lambda i: (0, 0)),
            pl.BlockSpec((1, HIDDEN), lambda i: (0, 0)),
            pl.BlockSpec((VTILE, HIDDEN), lambda i: (jnp.minimum(i, NT - 1), 0)),
            pl.BlockSpec((NT, VTILE), lambda i: (0, 0)),
        ],
        out_specs=pl.BlockSpec((NT, VTILE), lambda i: (0, 0)),
        out_shape=jax.ShapeDtypeStruct((NT, VTILE), jnp.float32),
        scratch_shapes=[
            pltpu.VMEM((1, HIDDEN), jnp.float32),
        ],
    )(emb_flat, W1, b1_2d, W2, b2_2d)


def kernel(inputs, emb, W1, b1, W2, b2):
    emb_flat, b2_2d = _sc_gather(inputs, emb.T, b2)
    out = _mlp(emb_flat, W1, b1.reshape(1, HIDDEN), W2, b2_2d)
    return out.reshape(1, VOCAB)
